# baseline (speedup 1.0000x reference)
import functools

import jax
import jax.numpy as jnp
from jax import lax
from jax.experimental import pallas as pl
from jax.experimental.pallas import tpu as pltpu
from jax.experimental.pallas import tpu_sc as plsc

_NC = 2
_NS = 16
_L = 16
_NW = _NC * _NS


def _dense_body(out_ref, w_ref, s_ref):
    w = jax.nn.softmax(w_ref[...])
    s_ref[...] = jnp.sum(out_ref[...] * w[None, :], axis=1)


def kernel(output_idx, output, history, assist_rate, assist_weight):
    B, NO = output.shape
    bpw = B // _NW
    groups = bpw // _L

    s = pl.pallas_call(
        _dense_body,
        out_shape=jax.ShapeDtypeStruct((B,), jnp.float32),
    )(output, assist_weight)

    mesh = plsc.VectorSubcoreMesh(core_axis_name="c", subcore_axis_name="s")

    @functools.partial(
        pl.kernel,
        out_type=jax.ShapeDtypeStruct((B,), jnp.float32),
        mesh=mesh,
        scratch_types=[
            pltpu.VMEM((bpw,), jnp.int32),
            pltpu.VMEM((bpw,), jnp.float32),
            pltpu.VMEM((bpw,), jnp.float32),
            pltpu.VMEM((bpw,), jnp.float32),
            pltpu.VMEM((bpw,), jnp.float32),
            pltpu.SemaphoreType.DMA,
        ],
    )
    def _combine_sc(idx_hbm, s_hbm, hist_hbm, rate_hbm, tgt_hbm,
                    idx_v, ar_v, s_v, hist_v, res_v, sem_ar):
        wid = lax.axis_index("s") * _NC + lax.axis_index("c")
        base = wid * bpw

        pltpu.sync_copy(idx_hbm.at[pl.ds(base, bpw)], idx_v)
        cp_ar = pltpu.async_copy(rate_hbm.at[idx_v], ar_v, sem_ar)
        pltpu.sync_copy(s_hbm.at[pl.ds(base, bpw)], s_v)
        pltpu.sync_copy(hist_hbm.at[pl.ds(base, bpw)], hist_v)
        cp_ar.wait()

        def body(g, carry):
            off = g * _L
            res_v[pl.ds(off, _L)] = (hist_v[pl.ds(off, _L)]
                                     + ar_v[pl.ds(off, _L)] * s_v[pl.ds(off, _L)])
            return carry

        lax.fori_loop(0, groups, body, 0)
        pltpu.sync_copy(res_v, tgt_hbm.at[pl.ds(base, bpw)])

    return _combine_sc(output_idx.astype(jnp.int32), s, history, assist_rate)
